# Initial kernel scaffold; baseline (speedup 1.0000x reference)
import jax
import jax.numpy as jnp
from jax.experimental import pallas as pl


def kernel(x, edge_attr, edge_index, batch, params):
    raise NotImplementedError("write your pallas kernel here")


# speedup vs baseline: 2.0365x; 2.0365x over previous
import functools

import jax
import jax.numpy as jnp
from jax import lax
from jax.experimental import pallas as pl
from jax.experimental.pallas import tpu as pltpu, tpu_sc as plsc

N = 50000
E = 800000
G = 64
NODE_IN = 128
EDGE_IN = 16
H = 64
NC_OUT = 5
NL = 3
BN_EPS = 1e-5

NP = 50176
EP = 802816
NBLK = 1024
EBLK = 1024
N_GRID = NP // NBLK
E_GRID = EP // EBLK
HH = H // 2

NUM_TILES = 16
CHUNK = 256
GROUPS_PER_CHUNK = CHUNK // 128
CHUNKS_PER_TILE = EP // (NUM_TILES * CHUNK)
ROWS_PER_TILE = NP // NUM_TILES


def _proj_node_body(x_ref, w_ref, b_ref, h_ref):
    h = jnp.dot(x_ref[...], w_ref[...], preferred_element_type=jnp.float32)
    h = h + b_ref[...]
    h_ref[0] = h[:, :HH]
    h_ref[1] = h[:, HH:]


def _proj_node(x_p, node_w, node_b):
    return pl.pallas_call(
        _proj_node_body,
        grid=(N_GRID,),
        in_specs=[
            pl.BlockSpec((NBLK, NODE_IN), lambda i: (i, 0)),
            pl.BlockSpec((NODE_IN, H), lambda i: (0, 0)),
            pl.BlockSpec((1, H), lambda i: (0, 0)),
        ],
        out_specs=pl.BlockSpec((2, NBLK, HH), lambda i: (0, i, 0)),
        out_shape=jax.ShapeDtypeStruct((2, NP, HH), jnp.float32),
    )(x_p, node_w, node_b.reshape(1, H))


def _proj_edge_body(ea_ref, ew_ref, eb_ref, lw_ref, lb_ref,
                    e0_ref, e1_ref, e2_ref):
    wc = jnp.dot(ew_ref[...], lw_ref[...], preferred_element_type=jnp.float32)
    bc = jnp.dot(eb_ref[...], lw_ref[...],
                 preferred_element_type=jnp.float32) + lb_ref[...]
    ee = jnp.dot(ea_ref[...], wc, preferred_element_type=jnp.float32) + bc
    for l, ref in enumerate((e0_ref, e1_ref, e2_ref)):
        ref[0] = ee[:, l * H:l * H + HH]
        ref[1] = ee[:, l * H + HH:(l + 1) * H]


def _proj_edge(ea_p, edge_w, edge_b, lw_all, lb_all):
    out_sds = jax.ShapeDtypeStruct((2, EP, HH), jnp.float32)
    return pl.pallas_call(
        _proj_edge_body,
        grid=(E_GRID,),
        in_specs=[
            pl.BlockSpec((EBLK, EDGE_IN), lambda i: (i, 0)),
            pl.BlockSpec((EDGE_IN, H), lambda i: (0, 0)),
            pl.BlockSpec((1, H), lambda i: (0, 0)),
            pl.BlockSpec((H, NL * H), lambda i: (0, 0)),
            pl.BlockSpec((1, NL * H), lambda i: (0, 0)),
        ],
        out_specs=[pl.BlockSpec((2, EBLK, HH), lambda i: (0, i, 0))] * NL,
        out_shape=[out_sds] * NL,
    )(ea_p, edge_w, edge_b.reshape(1, H), lw_all, lb_all)


def _sc_msg_body(h_hbm, ee_hbm, src_hbm, dst_hbm, aggr_hbm,
                 accum, sidx, didx, rows, eebuf, sem):
    c = lax.axis_index("c")
    s = lax.axis_index("s")
    tile_base = s * ROWS_PER_TILE

    def zero_row(i, _):
        z = jnp.zeros((16,), jnp.float32)
        rows[i, pl.ds(0, 16)] = z
        rows[i, pl.ds(16, 16)] = z
        return 0
    lax.fori_loop(0, CHUNK, zero_row, 0)
    zcopy = 224
    for k in range(ROWS_PER_TILE // zcopy):
        pltpu.sync_copy(rows.at[pl.ds(0, zcopy)],
                        accum.at[pl.ds(tile_base + k * zcopy, zcopy)])
    plsc.subcore_barrier()

    chunk0 = s * CHUNKS_PER_TILE
    src_off = c * NP

    def do_chunk(t, _):
        gbase = (chunk0 + t) * GROUPS_PER_CHUNK
        ebase = (chunk0 + t) * CHUNK
        pltpu.sync_copy(src_hbm.at[pl.ds(gbase, GROUPS_PER_CHUNK)], sidx)
        pltpu.sync_copy(dst_hbm.at[pl.ds(gbase, GROUPS_PER_CHUNK)], didx)
        for j in range(GROUPS_PER_CHUNK):
            for k in range(8):
                sl = pl.ds(k * 16, 16)
                sidx[j, sl] = sidx[j, sl] + src_off
        cps = [pltpu.async_copy(h_hbm.at[sidx.at[j]],
                                rows.at[pl.ds(j * 128, 128)], sem)
               for j in range(GROUPS_PER_CHUNK)]
        pltpu.sync_copy(ee_hbm.at[c, pl.ds(ebase, CHUNK)], eebuf)
        for cp in cps:
            cp.wait()

        def msg_row(i, _):
            for half in (0, 16):
                sl = pl.ds(half, 16)
                v = eebuf[i, sl] + rows[i, sl]
                eebuf[i, sl] = jnp.maximum(v, 0.0)
            return 0
        lax.fori_loop(0, CHUNK, msg_row, 0)

        for j in range(GROUPS_PER_CHUNK):
            pltpu.sync_copy(eebuf.at[pl.ds(j * 128, 128)],
                            accum.at[didx.at[j]], add=True)
        return 0

    lax.fori_loop(0, CHUNKS_PER_TILE, do_chunk, 0)
    plsc.subcore_barrier()
    pltpu.sync_copy(accum.at[pl.ds(tile_base, ROWS_PER_TILE)],
                    aggr_hbm.at[c, pl.ds(tile_base, ROWS_PER_TILE)])


@jax.jit
def _sc_msg(h_flat, ee, src2d, dst2d):
    mesh = plsc.VectorSubcoreMesh(core_axis_name="c", subcore_axis_name="s",
                                  num_cores=2, num_subcores=NUM_TILES)
    f = functools.partial(
        pl.kernel,
        out_type=jax.ShapeDtypeStruct((2, NP, HH), jnp.float32),
        mesh=mesh,
        scratch_types=[
            pltpu.VMEM_SHARED((NP, HH), jnp.float32),
            pltpu.VMEM((GROUPS_PER_CHUNK, 128), jnp.int32),
            pltpu.VMEM((GROUPS_PER_CHUNK, 128), jnp.int32),
            pltpu.VMEM((CHUNK, HH), jnp.float32),
            pltpu.VMEM((CHUNK, HH), jnp.float32),
            pltpu.SemaphoreType.DMA,
        ],
        compiler_params=pltpu.CompilerParams(use_tc_tiling_on_sc=False),
    )(_sc_msg_body)
    return f(h_flat, ee, src2d, dst2d)


def _mlp_body(h_ref, a_ref, w1_ref, b1_ref, w2_ref, b2_ref, sc_ref, sb_ref,
              out_ref):
    z = jnp.concatenate([h_ref[0] + a_ref[0], h_ref[1] + a_ref[1]], axis=1)
    t = jnp.maximum(
        jnp.dot(z, w1_ref[...], preferred_element_type=jnp.float32)
        + b1_ref[...], 0.0)
    t = jnp.dot(t, w2_ref[...], preferred_element_type=jnp.float32) + b2_ref[...]
    t = t * sc_ref[...] + sb_ref[...]
    t = jnp.maximum(t, 0.0)
    out_ref[0] = t[:, :HH]
    out_ref[1] = t[:, HH:]


def _node_mlp(h, aggr, w1, b1, w2, b2, scale, bias):
    wspec = pl.BlockSpec((H, H), lambda i: (0, 0))
    vspec = pl.BlockSpec((1, H), lambda i: (0, 0))
    return pl.pallas_call(
        _mlp_body,
        grid=(N_GRID,),
        in_specs=[
            pl.BlockSpec((2, NBLK, HH), lambda i: (0, i, 0)),
            pl.BlockSpec((2, NBLK, HH), lambda i: (0, i, 0)),
            wspec, vspec, wspec, vspec, vspec, vspec,
        ],
        out_specs=pl.BlockSpec((2, NBLK, HH), lambda i: (0, i, 0)),
        out_shape=jax.ShapeDtypeStruct((2, NP, HH), jnp.float32),
    )(h, aggr, w1, b1.reshape(1, H), w2, b2.reshape(1, H),
      scale.reshape(1, H), bias.reshape(1, H))


def _pool_body(h_ref, oh_ref, w1_ref, b1_ref, w2_ref, b2_ref, out_ref,
               sums_ref, maxes_ref, counts_ref):
    i = pl.program_id(0)

    @pl.when(i == 0)
    def _init():
        sums_ref[...] = jnp.zeros_like(sums_ref)
        counts_ref[...] = jnp.zeros_like(counts_ref)
        maxes_ref[...] = jnp.full_like(maxes_ref, -1e30)

    hcat = jnp.concatenate([h_ref[0], h_ref[1]], axis=1)
    oh = oh_ref[...]
    sums_ref[...] += lax.dot_general(
        oh, hcat, (((0,), (0,)), ((), ())),
        preferred_element_type=jnp.float32)
    counts_ref[...] += jnp.sum(oh, axis=0, keepdims=True)
    for g in range(G):
        m = jnp.where(oh[:, g:g + 1] > 0.0, hcat, -1e30)
        mg = jnp.max(m, axis=0, keepdims=True)
        maxes_ref[g:g + 1, :] = jnp.maximum(maxes_ref[g:g + 1, :], mg)

    @pl.when(i == pl.num_programs(0) - 1)
    def _final():
        counts = counts_ref[...].reshape(G, 1)
        sums = sums_ref[...]
        mean = sums / jnp.maximum(counts, 1.0)
        hmax = jnp.where(counts > 0.0, maxes_ref[...], 0.0)
        feat = jnp.concatenate([mean, hmax, sums], axis=1)
        r = jnp.maximum(
            jnp.dot(feat, w1_ref[...], preferred_element_type=jnp.float32)
            + b1_ref[...], 0.0)
        out_ref[...] = (jnp.dot(r, w2_ref[...],
                                preferred_element_type=jnp.float32)
                        + b2_ref[...])


def _pool_head(h, onehot, w1, b1, w2, b2):
    return pl.pallas_call(
        _pool_body,
        grid=(N_GRID,),
        in_specs=[
            pl.BlockSpec((2, NBLK, HH), lambda i: (0, i, 0)),
            pl.BlockSpec((NBLK, G), lambda i: (i, 0)),
            pl.BlockSpec((3 * H, H), lambda i: (0, 0)),
            pl.BlockSpec((1, H), lambda i: (0, 0)),
            pl.BlockSpec((H, NC_OUT), lambda i: (0, 0)),
            pl.BlockSpec((1, NC_OUT), lambda i: (0, 0)),
        ],
        out_specs=pl.BlockSpec((G, NC_OUT), lambda i: (0, 0)),
        out_shape=jax.ShapeDtypeStruct((G, NC_OUT), jnp.float32),
        scratch_shapes=[
            pltpu.VMEM((G, H), jnp.float32),
            pltpu.VMEM((G, H), jnp.float32),
            pltpu.VMEM((1, G), jnp.float32),
        ],
    )(h, onehot, w1, b1.reshape(1, H), w2, b2.reshape(1, NC_OUT))


def kernel(x, edge_attr, edge_index, batch, params):
    x_p = jnp.pad(x, ((0, NP - N), (0, 0)))
    ea_p = jnp.pad(edge_attr, ((0, EP - E), (0, 0)))
    src2d = jnp.pad(edge_index[0], (0, EP - E)).reshape(EP // 128, 128)
    dst2d = jnp.pad(edge_index[1], (0, EP - E),
                    constant_values=N).reshape(EP // 128, 128)
    onehot = (jnp.pad(batch, (0, NP - N), constant_values=G)[:, None]
              == jnp.arange(G, dtype=jnp.int32)[None, :]).astype(jnp.float32)

    lw_all = jnp.concatenate([lp['lin_e_w'] for lp in params['layers']], axis=1)
    lb_all = jnp.concatenate([lp['lin_e_b'] for lp in params['layers']]
                             ).reshape(1, NL * H)

    h = _proj_node(x_p, params['node_w'], params['node_b'])
    ees = _proj_edge(ea_p, params['edge_w'], params['edge_b'], lw_all, lb_all)

    inv = 1.0 / jnp.sqrt(1.0 + BN_EPS)
    for l, lp in enumerate(params['layers']):
        aggr = _sc_msg(h.reshape(2 * NP, HH), ees[l], src2d, dst2d)
        h = _node_mlp(h, aggr, lp['mlp_w1'], lp['mlp_b1'],
                      lp['mlp_w2'], lp['mlp_b2'],
                      lp['bn_g'] * inv, lp['bn_b'])

    return _pool_head(h, onehot, params['head_w1'], params['head_b1'],
                      params['head_w2'], params['head_b2'])
